# baseline (speedup 1.0000x reference)
import jax
import jax.numpy as jnp
from jax.experimental import pallas as pl
from jax.experimental.pallas import tpu as pltpu

_K, _N, _D_IN, _D_OUT = 2, 4096, 256, 256
_CH = 512
_NCH = _N // _CH
_NBUF = 3


def _gcn_pipeline(inp_hbm, adj_hbm, w_ref, out_hbm, s_ref, ibuf, abuf, obuf,
                  isem, insem, outsem):
    def in_copy(c, slot):
        return pltpu.make_async_copy(
            adj_hbm.at[pl.ds(c * _CH, _CH), :], abuf.at[slot], insem.at[slot])

    def out_copy(c, slot):
        return pltpu.make_async_copy(
            obuf.at[slot], out_hbm.at[:, pl.ds(c * _CH, _CH), :],
            outsem.at[slot])

    inp_cp = pltpu.make_async_copy(inp_hbm, ibuf, isem)
    inp_cp.start()
    for b in range(_NBUF):
        in_copy(b, b).start()

    inp_cp.wait()
    w = w_ref[...].astype(jnp.bfloat16)
    for k in range(_K):
        xk = ibuf[k].astype(jnp.bfloat16)
        sk = jnp.dot(xk, w, preferred_element_type=jnp.float32)
        s_ref[:, k * _D_OUT:(k + 1) * _D_OUT] = sk.astype(jnp.bfloat16)

    for c in range(_NCH):
        slot = c % _NBUF
        in_copy(c, slot).wait()
        if c >= _NBUF:
            out_copy(c - _NBUF, slot).wait()
        o = jnp.dot(abuf[slot], s_ref[...], preferred_element_type=jnp.float32)
        o = jnp.maximum(o, 0.0)
        for k in range(_K):
            obuf[slot, k] = o[:, k * _D_OUT:(k + 1) * _D_OUT]
        out_copy(c, slot).start()
        nxt = c + _NBUF
        if nxt < _NCH:
            in_copy(nxt, slot).start()

    for c in range(_NCH - _NBUF, _NCH):
        out_copy(c, c % _NBUF).wait()


def kernel(input, adj, weight):
    return pl.pallas_call(
        _gcn_pipeline,
        in_specs=[
            pl.BlockSpec(memory_space=pltpu.MemorySpace.HBM),
            pl.BlockSpec(memory_space=pltpu.MemorySpace.HBM),
            pl.BlockSpec(memory_space=pltpu.MemorySpace.VMEM),
        ],
        out_specs=pl.BlockSpec(memory_space=pltpu.MemorySpace.HBM),
        out_shape=jax.ShapeDtypeStruct((_K, _N, _D_OUT), jnp.float32),
        scratch_shapes=[
            pltpu.VMEM((_N, _K * _D_OUT), jnp.bfloat16),
            pltpu.VMEM((_K, _N, _D_IN), jnp.float32),
            pltpu.VMEM((_NBUF, _CH, _N), jnp.float32),
            pltpu.VMEM((_NBUF, _K, _CH, _D_OUT), jnp.float32),
            pltpu.SemaphoreType.DMA,
            pltpu.SemaphoreType.DMA((_NBUF,)),
            pltpu.SemaphoreType.DMA((_NBUF,)),
        ],
    )(input, adj, weight)


# speedup vs baseline: 1.0639x; 1.0639x over previous
import jax
import jax.numpy as jnp
from jax.experimental import pallas as pl
from jax.experimental.pallas import tpu as pltpu

_K, _N, _D_IN, _D_OUT = 2, 4096, 256, 256
_CH = 256
_NCH = _N // _CH
_NBUF = 6


def _gcn_pipeline(inp_ref, adj_hbm, w_ref, out_hbm, s_ref, abuf, obuf,
                  insem, outsem):
    def in_copy(c, slot):
        return pltpu.make_async_copy(
            adj_hbm.at[pl.ds(c * _CH, _CH), :], abuf.at[slot], insem.at[slot])

    def out_copy(c, slot):
        return pltpu.make_async_copy(
            obuf.at[slot], out_hbm.at[:, pl.ds(c * _CH, _CH), :],
            outsem.at[slot])

    for b in range(_NBUF):
        in_copy(b, b).start()

    w = w_ref[...].astype(jnp.bfloat16)
    for k in range(_K):
        xk = inp_ref[k].astype(jnp.bfloat16)
        sk = jnp.dot(xk, w, preferred_element_type=jnp.float32)
        s_ref[:, k * _D_OUT:(k + 1) * _D_OUT] = sk.astype(jnp.bfloat16)

    for c in range(_NCH):
        slot = c % _NBUF
        in_copy(c, slot).wait()
        if c >= _NBUF:
            out_copy(c - _NBUF, slot).wait()
        o = jnp.dot(abuf[slot], s_ref[...], preferred_element_type=jnp.float32)
        o = jnp.maximum(o, 0.0)
        for k in range(_K):
            obuf[slot, k] = o[:, k * _D_OUT:(k + 1) * _D_OUT]
        out_copy(c, slot).start()
        nxt = c + _NBUF
        if nxt < _NCH:
            in_copy(nxt, slot).start()

    for c in range(_NCH - _NBUF, _NCH):
        out_copy(c, c % _NBUF).wait()


def kernel(input, adj, weight):
    return pl.pallas_call(
        _gcn_pipeline,
        in_specs=[
            pl.BlockSpec(memory_space=pltpu.MemorySpace.VMEM),
            pl.BlockSpec(memory_space=pltpu.MemorySpace.HBM),
            pl.BlockSpec(memory_space=pltpu.MemorySpace.VMEM),
        ],
        out_specs=pl.BlockSpec(memory_space=pltpu.MemorySpace.HBM),
        out_shape=jax.ShapeDtypeStruct((_K, _N, _D_OUT), jnp.float32),
        scratch_shapes=[
            pltpu.VMEM((_N, _K * _D_OUT), jnp.bfloat16),
            pltpu.VMEM((_NBUF, _CH, _N), jnp.float32),
            pltpu.VMEM((_NBUF, _K, _CH, _D_OUT), jnp.float32),
            pltpu.SemaphoreType.DMA((_NBUF,)),
            pltpu.SemaphoreType.DMA((_NBUF,)),
        ],
    )(input, adj, weight)
